# Initial kernel scaffold; baseline (speedup 1.0000x reference)
import jax
import jax.numpy as jnp
from jax.experimental import pallas as pl


def kernel(gg_nfeat, gg_efeat, gg_edge_index, lg_nfeat, lg_efeat, lg_edge_index, l1_Wn, l1_bn, l1_Wni, l1_Wfij, l1_Wnj, l1_attn, l1_bias, l2_Wn, l2_bn, l2_Wni, l2_Wfij, l2_Wnj, l2_attn, l2_bias, ro_W1, ro_b1, ro_W2, ro_b2):
    raise NotImplementedError("write your pallas kernel here")


# speedup vs baseline: 4.7202x; 4.7202x over previous
import functools

import numpy as np
import jax
import jax.numpy as jnp
from jax import lax
from jax.experimental import pallas as pl
from jax.experimental.pallas import tpu as pltpu
from jax.experimental.pallas import tpu_sc as plsc

H = 8
N_GG, E_GG, N_LG, E_LG = 20000, 160000, 10000, 40000
NG_PAD, EG_PAD = 20480, 163840
NL_PAD, EL_PAD = 10240, 40960
NC, NS = 2, 16
NW = NC * NS

f32 = jnp.float32
i32 = jnp.int32


def _sc_mesh():
    return plsc.VectorSubcoreMesh(
        core_axis_name="c", subcore_axis_name="s", num_cores=NC, num_subcores=NS)


def _sc_gather(table, idx, chunk):
    N, D = table.shape
    E = idx.shape[0]
    per_w = E // NW
    n_ch = per_w // chunk
    assert per_w % chunk == 0 and chunk % 8 == 0 and chunk <= 128

    @functools.partial(
        pl.kernel,
        out_type=jax.ShapeDtypeStruct((E, D), f32),
        mesh=_sc_mesh(),
        scratch_types=[
            pltpu.VMEM((chunk,), i32),
            pltpu.VMEM((chunk, D), f32),
            pltpu.SemaphoreType.DMA,
        ],
    )
    def k(table_hbm, idx_hbm, out_hbm, idx_v, rows_v, sem):
        wid = lax.axis_index("s") * NC + lax.axis_index("c")

        def body(ci, carry):
            base = wid * per_w + ci * chunk
            pltpu.sync_copy(idx_hbm.at[pl.ds(base, chunk)], idx_v)
            pltpu.async_copy(table_hbm.at[idx_v], rows_v, sem).wait()
            pltpu.sync_copy(rows_v, out_hbm.at[pl.ds(base, chunk)])
            return carry

        lax.fori_loop(0, n_ch, body, 0)

    return k(table, idx)


def _sc_scatter_add(vals, idx, n_seg, chunk):
    E, D = vals.shape
    per_w = E // NW
    n_ch = per_w // chunk
    rpt = n_seg // NS
    assert per_w % chunk == 0 and n_seg % NS == 0
    zeros = jnp.zeros((rpt, D), f32)

    @functools.partial(
        pl.kernel,
        out_type=jax.ShapeDtypeStruct((NC * n_seg, D), f32),
        mesh=_sc_mesh(),
        scratch_types=[
            pltpu.VMEM((chunk,), i32),
            pltpu.VMEM((chunk, D), f32),
            pltpu.VMEM_SHARED((n_seg, D), f32),
        ],
    )
    def k(vals_hbm, idx_hbm, z_hbm, out_hbm, idx_v, vals_v, acc):
        cid = lax.axis_index("c")
        sid = lax.axis_index("s")
        wid = sid * NC + cid
        pltpu.sync_copy(z_hbm, acc.at[pl.ds(sid * rpt, rpt)])
        plsc.subcore_barrier()

        def body(ci, carry):
            base = wid * per_w + ci * chunk
            pltpu.sync_copy(idx_hbm.at[pl.ds(base, chunk)], idx_v)
            pltpu.sync_copy(vals_hbm.at[pl.ds(base, chunk)], vals_v)
            pltpu.sync_copy(vals_v, acc.at[idx_v], add=True)
            return carry

        lax.fori_loop(0, n_ch, body, 0)
        plsc.subcore_barrier()
        pltpu.sync_copy(acc.at[pl.ds(sid * rpt, rpt)],
                        out_hbm.at[pl.ds(cid * n_seg + sid * rpt, rpt)])

    return k(vals, idx, zeros)


def _sc_wsum(table_half, af_l, af_r, src, dst, n_half, chunk):
    E = src.shape[0]
    per_w = E // NW
    n_ch = per_w // chunk
    rpt = n_half // NS
    zeros = jnp.zeros((rpt, 128), f32)

    @functools.partial(
        pl.kernel,
        out_type=jax.ShapeDtypeStruct((NC * n_half, 128), f32),
        mesh=_sc_mesh(),
        scratch_types=[
            pltpu.VMEM((chunk,), i32),
            pltpu.VMEM((chunk,), i32),
            pltpu.VMEM((chunk,), i32),
            pltpu.VMEM((chunk * 8,), f32),
            pltpu.VMEM((chunk * 8,), f32),
            pltpu.VMEM((chunk, 512), f32),
            pltpu.VMEM((chunk, 128), f32),
            pltpu.VMEM_SHARED((n_half, 128), f32),
            pltpu.SemaphoreType.DMA,
        ],
    )
    def k(tab_hbm, afl_hbm, afr_hbm, src_hbm, dst_hbm, z_hbm, out_hbm,
          src_v, dst_v, idx2_v, afl_v, afr_v, rows_v, msg_v, acc, sem):
        cid = lax.axis_index("c")
        sid = lax.axis_index("s")
        wid = sid * NC + cid
        pltpu.sync_copy(z_hbm, acc.at[pl.ds(sid * rpt, rpt)])
        plsc.subcore_barrier()

        def body(ci, carry):
            base = wid * per_w + ci * chunk
            pltpu.sync_copy(src_hbm.at[pl.ds(base, chunk)], src_v)
            pltpu.sync_copy(dst_hbm.at[pl.ds(base, chunk)], dst_v)
            pltpu.sync_copy(afl_hbm.at[pl.ds(base * 8, chunk * 8)], afl_v)
            pltpu.sync_copy(afr_hbm.at[pl.ds(base * 8, chunk * 8)], afr_v)
            pltpu.async_copy(tab_hbm.at[src_v], rows_v, sem).wait()

            def halve(g, c2):
                d16 = dst_v[pl.ds(g * 16, 16)]
                idx2_v[pl.ds(g * 16, 16)] = lax.shift_right_logical(d16, 1)
                return c2

            lax.fori_loop(0, chunk // 16, halve, 0)

            def edge_pair(j, c2):
                avl = afl_v[pl.ds(j * 16, 16)]
                avr = afr_v[pl.ds(j * 16, 16)]
                for r in range(2):
                    i = j * 2 + r
                    accl = [jnp.zeros((16,), f32) for _ in range(4)]
                    accr = [jnp.zeros((16,), f32) for _ in range(4)]
                    for h in range(8):
                        sl = avl[r * 8 + h]
                        sr = avr[r * 8 + h]
                        for q in range(4):
                            row = rows_v[i, pl.ds(h * 64 + q * 16, 16)]
                            accl[q] = accl[q] + sl * row
                            accr[q] = accr[q] + sr * row
                    for q in range(4):
                        msg_v[i, pl.ds(q * 16, 16)] = accl[q]
                        msg_v[i, pl.ds(64 + q * 16, 16)] = accr[q]
                return c2

            lax.fori_loop(0, chunk // 2, edge_pair, 0)
            pltpu.sync_copy(msg_v, acc.at[idx2_v], add=True)
            return carry

        lax.fori_loop(0, n_ch, body, 0)
        plsc.subcore_barrier()
        pltpu.sync_copy(acc.at[pl.ds(sid * rpt, rpt)],
                        out_hbm.at[pl.ds(cid * n_half + sid * rpt, rpt)])

    return k(table_half, af_l, af_r, src, dst, zeros)


def _tc_matmul_multi(x, wbs, bm):
    m, kdim = x.shape
    grid = m // bm
    n = len(wbs)

    def body(*refs):
        xb = refs[0][...]
        for j in range(n):
            w = refs[1 + 2 * j][...]
            b = refs[2 + 2 * j][...]
            refs[1 + 2 * n + j][...] = (
                jnp.dot(xb, w, preferred_element_type=f32) + b)

    in_specs = [pl.BlockSpec((bm, kdim), lambda i: (i, 0))]
    ins = [x]
    for (w, b) in wbs:
        in_specs.append(pl.BlockSpec(w.shape, lambda i: (0, 0)))
        in_specs.append(pl.BlockSpec(b.shape, lambda i: (0, 0)))
        ins.extend([w, b])
    out_shape = [jax.ShapeDtypeStruct((m, w.shape[1]), f32) for (w, _) in wbs]
    out_specs = [pl.BlockSpec((bm, w.shape[1]), lambda i: (i, 0)) for (w, _) in wbs]
    return pl.pallas_call(body, grid=(grid,), in_specs=in_specs,
                          out_specs=out_specs, out_shape=out_shape)(*ins)


def _tc_edge_gg(g_ni, g_nj, ef, dst2d, wfij_t, a1, p36, bm=512):
    e_pad = g_ni.shape[0]
    grid = e_pad // bm

    def body(ni_ref, nj_ref, ef_ref, dst_ref, w_ref, a_ref, p_ref,
             ee_ref, idx_ref, hs_ref):
        i = pl.program_id(0)
        f = (ni_ref[...] + nj_ref[...]
             + jnp.dot(ef_ref[...], w_ref[...], preferred_element_type=f32))
        t = jnp.where(f > 0, f, 0.01 * f)
        e = jnp.dot(t, a_ref[...], preferred_element_type=f32)
        rows = i * bm + lax.broadcasted_iota(i32, (bm, H), 0)
        ee = jnp.where(rows < E_GG, jnp.exp(e), 0.0)
        dst = dst_ref[...]
        group = lax.broadcasted_iota(i32, (bm, 128), 1) // 8
        ee_ref[...] = jnp.tile(ee, (1, 16)) * (group == (dst % 16)).astype(f32)
        idx_ref[...] = dst // 16
        hs_ref[...] = jnp.dot(t, p_ref[...], preferred_element_type=f32)

    return pl.pallas_call(
        body, grid=(grid,),
        in_specs=[
            pl.BlockSpec((bm, 384), lambda i: (i, 0)),
            pl.BlockSpec((bm, 384), lambda i: (i, 0)),
            pl.BlockSpec((bm, 16), lambda i: (i, 0)),
            pl.BlockSpec((bm, 1), lambda i: (i, 0)),
            pl.BlockSpec((16, 384), lambda i: (0, 0)),
            pl.BlockSpec((384, H), lambda i: (0, 0)),
            pl.BlockSpec((384, 36), lambda i: (0, 0)),
        ],
        out_specs=[pl.BlockSpec((bm, 128), lambda i: (i, 0)),
                   pl.BlockSpec((bm, 1), lambda i: (i, 0)),
                   pl.BlockSpec((bm, 36), lambda i: (i, 0))],
        out_shape=[jax.ShapeDtypeStruct((e_pad, 128), f32),
                   jax.ShapeDtypeStruct((e_pad, 1), i32),
                   jax.ShapeDtypeStruct((e_pad, 36), f32)],
    )(g_ni, g_nj, ef, dst2d, wfij_t, a1, p36)


def _tc_combine_pad(den_p, n_seg, bm=512):
    grid = n_seg // bm
    nb = n_seg // bm

    def body(p0_ref, p1_ref, out_ref):
        s = p0_ref[...] + p1_ref[...] + 1e-30
        out_ref[...] = jnp.concatenate([s, jnp.zeros((bm, 120), f32)], axis=1)

    return pl.pallas_call(
        body, grid=(grid,),
        in_specs=[pl.BlockSpec((bm, H), lambda i: (i, 0)),
                  pl.BlockSpec((bm, H), lambda i: (i + nb, 0))],
        out_specs=pl.BlockSpec((bm, 128), lambda i: (i, 0)),
        out_shape=jax.ShapeDtypeStruct((n_seg, 128), f32),
    )(den_p, den_p)


def _tc_edge_lg(lg_ef, y2c0, y2c1, g_ni, g_nj, dst2d, wfij_t, a2, bm=512):
    e_pad = lg_ef.shape[0]
    grid = e_pad // bm
    hb = bm // 2

    def body(ef_ref, p00, p01, p10, p11, ni_ref, nj_ref, dst_ref,
             w_ref, a_ref, ee_ref, idx_ref):
        i = pl.program_id(0)
        y2 = jnp.concatenate([p00[...] + p01[...], p10[...] + p11[...]], axis=1)
        brep = jnp.repeat(y2, 2, axis=0)
        f = (ni_ref[...] + nj_ref[...]
             + jnp.dot(brep + ef_ref[...], w_ref[...], preferred_element_type=f32))
        t = jnp.where(f > 0, f, 0.01 * f)
        e = jnp.dot(t, a_ref[...], preferred_element_type=f32)
        rows = i * bm + lax.broadcasted_iota(i32, (bm, H), 0)
        ee = jnp.where(rows < E_LG, jnp.exp(e), 0.0)
        dst = dst_ref[...]
        group = lax.broadcasted_iota(i32, (bm, 128), 1) // 8
        ee_ref[...] = jnp.tile(ee, (1, 16)) * (group == (dst % 16)).astype(f32)
        idx_ref[...] = dst // 16

    ng_blocks = NG_PAD // hb
    return pl.pallas_call(
        body, grid=(grid,),
        in_specs=[
            pl.BlockSpec((bm, 128), lambda i: (i, 0)),
            pl.BlockSpec((hb, 64), lambda i: (i, 0)),
            pl.BlockSpec((hb, 64), lambda i: (i + ng_blocks, 0)),
            pl.BlockSpec((hb, 64), lambda i: (i, 0)),
            pl.BlockSpec((hb, 64), lambda i: (i + ng_blocks, 0)),
            pl.BlockSpec((bm, 1024), lambda i: (i, 0)),
            pl.BlockSpec((bm, 1024), lambda i: (i, 0)),
            pl.BlockSpec((bm, 1), lambda i: (i, 0)),
            pl.BlockSpec((128, 1024), lambda i: (0, 0)),
            pl.BlockSpec((1024, H), lambda i: (0, 0)),
        ],
        out_specs=[pl.BlockSpec((bm, 128), lambda i: (i, 0)),
                   pl.BlockSpec((bm, 1), lambda i: (i, 0))],
        out_shape=[jax.ShapeDtypeStruct((e_pad, 128), f32),
                   jax.ShapeDtypeStruct((e_pad, 1), i32)],
    )(lg_ef, y2c0, y2c0, y2c1, y2c1, g_ni, g_nj, dst2d, wfij_t, a2)


def _tc_softmax_div(ee_spread, g_den, dst2d, bm=1024):
    e_pad = ee_spread.shape[0]
    grid = e_pad // bm

    def body(ee_ref, d_ref, dst_ref, al_ref, ar_ref):
        ee = jnp.sum(ee_ref[...].reshape(bm, 16, H), axis=1)
        a = ee / d_ref[...][:, :H]
        par = (dst_ref[...] % 2).astype(f32)
        al_ref[...] = a * (1.0 - par)
        ar_ref[...] = a * par

    return pl.pallas_call(
        body, grid=(grid,),
        in_specs=[pl.BlockSpec((bm, 128), lambda i: (i, 0)),
                  pl.BlockSpec((bm, 128), lambda i: (i, 0)),
                  pl.BlockSpec((bm, 1), lambda i: (i, 0))],
        out_specs=[pl.BlockSpec((bm, H), lambda i: (i, 0)),
                   pl.BlockSpec((bm, H), lambda i: (i, 0))],
        out_shape=[jax.ShapeDtypeStruct((e_pad, H), f32),
                   jax.ShapeDtypeStruct((e_pad, H), f32)],
    )(ee_spread, g_den, dst2d)


def _tc_softmax_div_spread(ee_spread, g_den, src2d, bm=1024):
    e_pad = ee_spread.shape[0]
    grid = e_pad // bm

    def body(ee_ref, d_ref, src_ref, a_ref, idx_ref):
        ee = jnp.sum(ee_ref[...].reshape(bm, 16, H), axis=1)
        a = ee / d_ref[...][:, :H]
        src = src_ref[...]
        group = lax.broadcasted_iota(i32, (bm, 128), 1) // 8
        a_ref[...] = jnp.tile(a, (1, 16)) * (group == (src % 16)).astype(f32)
        idx_ref[...] = src // 16

    return pl.pallas_call(
        body, grid=(grid,),
        in_specs=[pl.BlockSpec((bm, 128), lambda i: (i, 0)),
                  pl.BlockSpec((bm, 128), lambda i: (i, 0)),
                  pl.BlockSpec((bm, 1), lambda i: (i, 0))],
        out_specs=[pl.BlockSpec((bm, 128), lambda i: (i, 0)),
                   pl.BlockSpec((bm, 1), lambda i: (i, 0))],
        out_shape=[jax.ShapeDtypeStruct((e_pad, 128), f32),
                   jax.ShapeDtypeStruct((e_pad, 1), i32)],
    )(ee_spread, g_den, src2d)


def _tc_final(bp, h2, hs36, ro_w1t, ro_b1, ro_w2t, ro_b2):
    grid = 20
    bn = NL_PAD // grid
    bh = N_GG // grid

    def body(b0_ref, b1_ref, h2_ref, hs_ref, w1_ref, b1v_ref, w2_ref, b2v_ref,
             out_ref, s1, s2):
        i = pl.program_id(0)

        @pl.when(i == 0)
        def _():
            s1[...] = jnp.zeros((1, 128), f32)
            s2[...] = jnp.zeros((1, 36), f32)

        b = b0_ref[...] + b1_ref[...]
        h2b = h2_ref[...].reshape(bn, H, 128)
        contrib = jnp.sum(b[:, :, None] * h2b, axis=(0, 1))
        s1[...] = s1[...] + contrib.reshape(1, 128)
        s2[...] = s2[...] + jnp.sum(hs_ref[...], axis=0).reshape(1, 36)

        @pl.when(i == grid - 1)
        def _():
            y1 = s1[0, :] * (1.0 / N_LG)
            y2p = s2[0, :] * (1.0 / N_GG)
            y = jnp.concatenate([y1, y2p])
            x = jax.nn.sigmoid(jnp.dot(y, w1_ref[...],
                                       preferred_element_type=f32) + b1v_ref[0, :])
            o = jax.nn.sigmoid(jnp.dot(x, w2_ref[...],
                                       preferred_element_type=f32)[0] + b2v_ref[0, 0])
            out_ref[...] = jnp.full((8, 128), o, f32)

    nl_blocks = NL_PAD // bn
    return pl.pallas_call(
        body, grid=(grid,),
        in_specs=[
            pl.BlockSpec((bn, H), lambda i: (i, 0)),
            pl.BlockSpec((bn, H), lambda i: (i + nl_blocks, 0)),
            pl.BlockSpec((bn, 1024), lambda i: (i, 0)),
            pl.BlockSpec((bh, 36), lambda i: (i, 0)),
            pl.BlockSpec((164, 128), lambda i: (0, 0)),
            pl.BlockSpec((1, 128), lambda i: (0, 0)),
            pl.BlockSpec((128, 1), lambda i: (0, 0)),
            pl.BlockSpec((1, 1), lambda i: (0, 0)),
        ],
        out_specs=pl.BlockSpec((8, 128), lambda i: (0, 0)),
        out_shape=jax.ShapeDtypeStruct((8, 128), f32),
        scratch_shapes=[pltpu.VMEM((1, 128), f32), pltpu.VMEM((1, 36), f32)],
    )(bp, bp, h2, hs36, ro_w1t, ro_b1, ro_w2t, ro_b2)


_PERM0 = np.add.outer(np.arange(H) * 128, np.arange(64)).reshape(-1)
_PERM1 = _PERM0 + 64
_P36_PAD = np.concatenate([np.tile(np.eye(36, dtype=np.float32), (H, 1)),
                           np.zeros((96, 36), np.float32)], axis=0)
_A1_ROWS = np.arange(288)
_A1_COLS = np.repeat(np.arange(H), 36)
_A2_ROWS = np.arange(1024)
_A2_COLS = np.repeat(np.arange(H), 128)


def kernel(gg_nfeat, gg_efeat, gg_edge_index, lg_nfeat, lg_efeat, lg_edge_index,
           l1_Wn, l1_bn, l1_Wni, l1_Wfij, l1_Wnj, l1_attn, l1_bias,
           l2_Wn, l2_bn, l2_Wni, l2_Wfij, l2_Wnj, l2_attn, l2_bias,
           ro_W1, ro_b1, ro_W2, ro_b2):
    src_g = jnp.concatenate([gg_edge_index[0], jnp.zeros(EG_PAD - E_GG, i32)])
    dst_g = jnp.concatenate([gg_edge_index[1], jnp.zeros(EG_PAD - E_GG, i32)])
    src_l = jnp.concatenate([lg_edge_index[0], jnp.zeros(EL_PAD - E_LG, i32)])
    dst_l = jnp.concatenate([lg_edge_index[1], jnp.zeros(EL_PAD - E_LG, i32)])
    gg_nf = jnp.concatenate([gg_nfeat, jnp.zeros((NG_PAD - N_GG, 128), f32)])
    lg_nf = jnp.concatenate([lg_nfeat, jnp.zeros((NL_PAD - N_LG, 128), f32)])
    gg_ef = jnp.concatenate([gg_efeat, jnp.zeros((EG_PAD - E_GG, 16), f32)])
    lg_ef = jnp.concatenate([lg_efeat, jnp.zeros((EL_PAD - E_LG, 128), f32)])

    wn1c0_t = l1_Wn[_PERM0, :].T
    wn1c1_t = l1_Wn[_PERM1, :].T
    bn1c0 = l1_bn[_PERM0].reshape(1, 512)
    bn1c1 = l1_bn[_PERM1].reshape(1, 512)
    a1 = jnp.zeros((384, H), f32).at[_A1_ROWS, _A1_COLS].set(l1_attn.reshape(-1))
    a2 = jnp.zeros((1024, H), f32).at[_A2_ROWS, _A2_COLS].set(l2_attn.reshape(-1))
    zero96 = jnp.zeros((1, 96), f32)
    zero1024 = jnp.zeros((1, 1024), f32)
    wni1_t = jnp.concatenate([l1_Wni.T, jnp.zeros((128, 96), f32)], axis=1)
    wnj1_t = jnp.concatenate([l1_Wnj.T, jnp.zeros((128, 96), f32)], axis=1)
    wfij1_t = jnp.concatenate([l1_Wfij.T, jnp.zeros((16, 96), f32)], axis=1)
    bias1 = jnp.concatenate([l1_bias.reshape(1, 288), zero96], axis=1)

    f_ni1, f_nj1, h1c0, h1c1 = _tc_matmul_multi(
        gg_nf,
        [(wni1_t, bias1),
         (wnj1_t, jnp.zeros((1, 384), f32)),
         (wn1c0_t, bn1c0),
         (wn1c1_t, bn1c1)],
        bm=512)

    dst_g2d = dst_g.reshape(-1, 1)
    g_ni1 = _sc_gather(f_ni1, src_g, 128)
    g_nj1 = _sc_gather(f_nj1, dst_g, 128)
    ee1s, idx16_g, hs36 = _tc_edge_gg(
        g_ni1, g_nj1, gg_ef, dst_g2d, wfij1_t, a1, jnp.asarray(_P36_PAD))

    n16g = NG_PAD // 16
    den1p = _sc_scatter_add(ee1s, idx16_g.reshape(-1), n16g, 128)
    den1_8 = jnp.concatenate([den1p[:n16g].reshape(NG_PAD, H),
                              den1p[n16g:].reshape(NG_PAD, H)], axis=0)
    den1_pad = _tc_combine_pad(den1_8, NG_PAD)
    g_den1 = _sc_gather(den1_pad, dst_g, 128)
    afl1, afr1 = _tc_softmax_div(ee1s, g_den1, dst_g2d)

    nhg = NG_PAD // 2
    y2c0p = _sc_wsum(h1c0, afl1.reshape(-1), afr1.reshape(-1),
                     src_g, dst_g, nhg, 64)
    y2c1p = _sc_wsum(h1c1, afl1.reshape(-1), afr1.reshape(-1),
                     src_g, dst_g, nhg, 64)
    y2c0 = jnp.concatenate([y2c0p[:nhg].reshape(NG_PAD, 64),
                            y2c0p[nhg:].reshape(NG_PAD, 64)], axis=0)
    y2c1 = jnp.concatenate([y2c1p[:nhg].reshape(NG_PAD, 64),
                            y2c1p[nhg:].reshape(NG_PAD, 64)], axis=0)

    f_ni2, f_nj2, h2 = _tc_matmul_multi(
        lg_nf,
        [(l2_Wni.T, l2_bias.reshape(1, 1024)),
         (l2_Wnj.T, zero1024),
         (l2_Wn.T, l2_bn.reshape(1, 1024))],
        bm=512)

    dst_l2d = dst_l.reshape(-1, 1)
    src_l2d = src_l.reshape(-1, 1)
    g_ni2 = _sc_gather(f_ni2, src_l, 64)
    g_nj2 = _sc_gather(f_nj2, dst_l, 64)
    ee2s, idx16_l = _tc_edge_lg(lg_ef, y2c0, y2c1, g_ni2, g_nj2, dst_l2d,
                                l2_Wfij.T, a2)

    n16l = NL_PAD // 16
    den2p = _sc_scatter_add(ee2s, idx16_l.reshape(-1), n16l, 128)
    den2_8 = jnp.concatenate([den2p[:n16l].reshape(NL_PAD, H),
                              den2p[n16l:].reshape(NL_PAD, H)], axis=0)
    den2_pad = _tc_combine_pad(den2_8, NL_PAD)
    g_den2 = _sc_gather(den2_pad, dst_l, 128)
    a2s, idx16_s = _tc_softmax_div_spread(ee2s, g_den2, src_l2d)

    bp128 = _sc_scatter_add(a2s, idx16_s.reshape(-1), n16l, 128)
    bp = jnp.concatenate([bp128[:n16l].reshape(NL_PAD, H),
                          bp128[n16l:].reshape(NL_PAD, H)], axis=0)

    out_pad = _tc_final(bp, h2, hs36, ro_W1.T, ro_b1.reshape(1, 128),
                        ro_W2.T, ro_b2.reshape(1, 1))
    return out_pad[0, 0:1]


# speedup vs baseline: 5.4788x; 1.1607x over previous
import functools

import numpy as np
import jax
import jax.numpy as jnp
from jax import lax
from jax.experimental import pallas as pl
from jax.experimental.pallas import tpu as pltpu
from jax.experimental.pallas import tpu_sc as plsc

H = 8
N_GG, E_GG, N_LG, E_LG = 20000, 160000, 10000, 40000
NG_PAD, EG_PAD = 20480, 163840
NL_PAD, EL_PAD = 10240, 40960
NC, NS = 2, 16
NW = NC * NS

f32 = jnp.float32
i32 = jnp.int32


def _sc_mesh():
    return plsc.VectorSubcoreMesh(
        core_axis_name="c", subcore_axis_name="s", num_cores=NC, num_subcores=NS)


def _sc_gather(table, idx, chunk):
    N, D = table.shape
    E = idx.shape[0]
    per_w = E // NW
    n_ch = per_w // chunk
    assert per_w % chunk == 0 and chunk % 8 == 0 and chunk <= 128
    assert n_ch % 2 == 0

    @functools.partial(
        pl.kernel,
        out_type=jax.ShapeDtypeStruct((E, D), f32),
        mesh=_sc_mesh(),
        scratch_types=[
            pltpu.VMEM((2, chunk), i32),
            pltpu.VMEM((2, chunk, D), f32),
            pltpu.SemaphoreType.DMA,
            pltpu.SemaphoreType.DMA,
        ],
    )
    def k(table_hbm, idx_hbm, out_hbm, idx_v, rows_v, s0, s1):
        wid = lax.axis_index("s") * NC + lax.axis_index("c")
        base0 = wid * per_w
        sems = (s0, s1)

        def fire(ci, b):
            pltpu.sync_copy(idx_hbm.at[pl.ds(base0 + ci * chunk, chunk)],
                            idx_v.at[b])
            pltpu.async_copy(table_hbm.at[idx_v.at[b]], rows_v.at[b], sems[b])

        fire(0, 0)

        def body(ci2, carry):
            for b in range(2):
                ci = ci2 * 2 + b

                @pl.when(ci + 1 < n_ch)
                def _():
                    fire(ci + 1, 1 - b)

                pltpu.make_async_copy(table_hbm.at[idx_v.at[b]],
                                      rows_v.at[b], sems[b]).wait()
                pltpu.sync_copy(rows_v.at[b],
                                out_hbm.at[pl.ds(base0 + ci * chunk, chunk)])
            return carry

        lax.fori_loop(0, n_ch // 2, body, 0)

    return k(table, idx)


def _sc_gather2(table_a, table_b, idx_a, idx_b, chunk):
    N, D = table_a.shape
    E = idx_a.shape[0]
    per_w = E // NW
    n_ch = per_w // chunk
    assert per_w % chunk == 0 and chunk % 8 == 0 and chunk <= 128
    assert n_ch % 2 == 0

    @functools.partial(
        pl.kernel,
        out_type=[jax.ShapeDtypeStruct((E, D), f32),
                  jax.ShapeDtypeStruct((E, D), f32)],
        mesh=_sc_mesh(),
        scratch_types=[
            pltpu.VMEM((2, chunk), i32),
            pltpu.VMEM((2, chunk), i32),
            pltpu.VMEM((2, chunk, D), f32),
            pltpu.VMEM((2, chunk, D), f32),
            pltpu.SemaphoreType.DMA,
            pltpu.SemaphoreType.DMA,
            pltpu.SemaphoreType.DMA,
            pltpu.SemaphoreType.DMA,
        ],
    )
    def k(ta_hbm, tb_hbm, ia_hbm, ib_hbm, oa_hbm, ob_hbm,
          ia_v, ib_v, ra_v, rb_v, sa0, sa1, sb0, sb1):
        wid = lax.axis_index("s") * NC + lax.axis_index("c")
        base0 = wid * per_w
        sa = (sa0, sa1)
        sb = (sb0, sb1)

        def fire(ci, b):
            sl = pl.ds(base0 + ci * chunk, chunk)
            pltpu.sync_copy(ia_hbm.at[sl], ia_v.at[b])
            pltpu.sync_copy(ib_hbm.at[sl], ib_v.at[b])
            pltpu.async_copy(ta_hbm.at[ia_v.at[b]], ra_v.at[b], sa[b])
            pltpu.async_copy(tb_hbm.at[ib_v.at[b]], rb_v.at[b], sb[b])

        fire(0, 0)

        def body(ci2, carry):
            for b in range(2):
                ci = ci2 * 2 + b

                @pl.when(ci + 1 < n_ch)
                def _():
                    fire(ci + 1, 1 - b)

                sl = pl.ds(base0 + ci * chunk, chunk)
                pltpu.make_async_copy(ta_hbm.at[ia_v.at[b]],
                                      ra_v.at[b], sa[b]).wait()
                pltpu.sync_copy(ra_v.at[b], oa_hbm.at[sl])
                pltpu.make_async_copy(tb_hbm.at[ib_v.at[b]],
                                      rb_v.at[b], sb[b]).wait()
                pltpu.sync_copy(rb_v.at[b], ob_hbm.at[sl])
            return carry

        lax.fori_loop(0, n_ch // 2, body, 0)

    return k(table_a, table_b, idx_a, idx_b)


def _sc_scatter_add(vals, idx, n_seg, chunk):
    E, D = vals.shape
    per_w = E // NW
    n_ch = per_w // chunk
    rpt = n_seg // NS
    assert per_w % chunk == 0 and n_seg % NS == 0
    zeros = jnp.zeros((rpt, D), f32)

    assert n_ch % 2 == 0

    @functools.partial(
        pl.kernel,
        out_type=jax.ShapeDtypeStruct((NC * n_seg, D), f32),
        mesh=_sc_mesh(),
        scratch_types=[
            pltpu.VMEM((2, chunk), i32),
            pltpu.VMEM((2, chunk, D), f32),
            pltpu.VMEM_SHARED((n_seg, D), f32),
            pltpu.SemaphoreType.DMA,
            pltpu.SemaphoreType.DMA,
        ],
    )
    def k(vals_hbm, idx_hbm, z_hbm, out_hbm, idx_v, vals_v, acc, s0, s1):
        cid = lax.axis_index("c")
        sid = lax.axis_index("s")
        wid = sid * NC + cid
        sems = (s0, s1)

        def fire(ci, b):
            base = wid * per_w + ci * chunk
            pltpu.sync_copy(idx_hbm.at[pl.ds(base, chunk)], idx_v.at[b])
            pltpu.async_copy(vals_hbm.at[pl.ds(base, chunk)], vals_v.at[b],
                             sems[b])

        pltpu.sync_copy(z_hbm, acc.at[pl.ds(sid * rpt, rpt)])
        plsc.subcore_barrier()
        fire(0, 0)

        def body(ci2, carry):
            for b in range(2):
                ci = ci2 * 2 + b

                @pl.when(ci + 1 < n_ch)
                def _():
                    fire(ci + 1, 1 - b)

                base = wid * per_w + ci * chunk
                pltpu.make_async_copy(vals_hbm.at[pl.ds(base, chunk)],
                                      vals_v.at[b], sems[b]).wait()
                pltpu.sync_copy(vals_v.at[b], acc.at[idx_v.at[b]], add=True)
            return carry

        lax.fori_loop(0, n_ch // 2, body, 0)
        plsc.subcore_barrier()
        pltpu.sync_copy(acc.at[pl.ds(sid * rpt, rpt)],
                        out_hbm.at[pl.ds(cid * n_seg + sid * rpt, rpt)])

    return k(vals, idx, zeros)


def _sc_wsum(table_half, af_l, af_r, src, dst, n_half, chunk):
    E = src.shape[0]
    per_w = E // NW
    n_ch = per_w // chunk
    rpt = n_half // NS
    zeros = jnp.zeros((rpt, 128), f32)

    assert n_ch % 2 == 0

    @functools.partial(
        pl.kernel,
        out_type=jax.ShapeDtypeStruct((NC * n_half, 128), f32),
        mesh=_sc_mesh(),
        scratch_types=[
            pltpu.VMEM((2, chunk), i32),
            pltpu.VMEM((2, chunk), i32),
            pltpu.VMEM((chunk,), i32),
            pltpu.VMEM((2, chunk * 8), f32),
            pltpu.VMEM((2, chunk * 8), f32),
            pltpu.VMEM((2, chunk, 512), f32),
            pltpu.VMEM((chunk, 128), f32),
            pltpu.VMEM_SHARED((n_half, 128), f32),
            pltpu.SemaphoreType.DMA,
            pltpu.SemaphoreType.DMA,
        ],
    )
    def k(tab_hbm, afl_hbm, afr_hbm, src_hbm, dst_hbm, z_hbm, out_hbm,
          src_v, dst_v, idx2_v, afl_v, afr_v, rows_v, msg_v, acc, s0, s1):
        cid = lax.axis_index("c")
        sid = lax.axis_index("s")
        wid = sid * NC + cid
        sems = (s0, s1)

        def fire(ci, b):
            base = wid * per_w + ci * chunk
            pltpu.sync_copy(src_hbm.at[pl.ds(base, chunk)], src_v.at[b])
            pltpu.sync_copy(dst_hbm.at[pl.ds(base, chunk)], dst_v.at[b])
            pltpu.sync_copy(afl_hbm.at[pl.ds(base * 8, chunk * 8)], afl_v.at[b])
            pltpu.sync_copy(afr_hbm.at[pl.ds(base * 8, chunk * 8)], afr_v.at[b])
            pltpu.async_copy(tab_hbm.at[src_v.at[b]], rows_v.at[b], sems[b])

        pltpu.sync_copy(z_hbm, acc.at[pl.ds(sid * rpt, rpt)])
        plsc.subcore_barrier()
        fire(0, 0)

        def body(ci2, carry):
            for b in range(2):
                ci = ci2 * 2 + b

                @pl.when(ci + 1 < n_ch)
                def _():
                    fire(ci + 1, 1 - b)

                pltpu.make_async_copy(tab_hbm.at[src_v.at[b]],
                                      rows_v.at[b], sems[b]).wait()

                def halve(g, c2):
                    d16 = dst_v[b, pl.ds(g * 16, 16)]
                    idx2_v[pl.ds(g * 16, 16)] = lax.shift_right_logical(d16, 1)
                    return c2

                lax.fori_loop(0, chunk // 16, halve, 0)

                def edge_pair(j, c2):
                    avl = afl_v[b, pl.ds(j * 16, 16)]
                    avr = afr_v[b, pl.ds(j * 16, 16)]
                    for r in range(2):
                        i = j * 2 + r
                        accl = [jnp.zeros((16,), f32) for _ in range(4)]
                        accr = [jnp.zeros((16,), f32) for _ in range(4)]
                        for h in range(8):
                            sl = avl[r * 8 + h]
                            sr = avr[r * 8 + h]
                            for q in range(4):
                                row = rows_v[b, i, pl.ds(h * 64 + q * 16, 16)]
                                accl[q] = accl[q] + sl * row
                                accr[q] = accr[q] + sr * row
                        for q in range(4):
                            msg_v[i, pl.ds(q * 16, 16)] = accl[q]
                            msg_v[i, pl.ds(64 + q * 16, 16)] = accr[q]
                    return c2

                lax.fori_loop(0, chunk // 2, edge_pair, 0)
                pltpu.sync_copy(msg_v, acc.at[idx2_v], add=True)
            return carry

        lax.fori_loop(0, n_ch // 2, body, 0)
        plsc.subcore_barrier()
        pltpu.sync_copy(acc.at[pl.ds(sid * rpt, rpt)],
                        out_hbm.at[pl.ds(cid * n_half + sid * rpt, rpt)])

    return k(table_half, af_l, af_r, src, dst, zeros)


def _tc_matmul_multi(x, wbs, bm):
    m, kdim = x.shape
    grid = m // bm
    n = len(wbs)

    def body(*refs):
        xb = refs[0][...]
        for j in range(n):
            w = refs[1 + 2 * j][...]
            b = refs[2 + 2 * j][...]
            refs[1 + 2 * n + j][...] = (
                jnp.dot(xb, w, preferred_element_type=f32) + b)

    in_specs = [pl.BlockSpec((bm, kdim), lambda i: (i, 0))]
    ins = [x]
    for (w, b) in wbs:
        in_specs.append(pl.BlockSpec(w.shape, lambda i: (0, 0)))
        in_specs.append(pl.BlockSpec(b.shape, lambda i: (0, 0)))
        ins.extend([w, b])
    out_shape = [jax.ShapeDtypeStruct((m, w.shape[1]), f32) for (w, _) in wbs]
    out_specs = [pl.BlockSpec((bm, w.shape[1]), lambda i: (i, 0)) for (w, _) in wbs]
    return pl.pallas_call(body, grid=(grid,), in_specs=in_specs,
                          out_specs=out_specs, out_shape=out_shape)(*ins)


def _tc_edge_gg(g_ni, g_nj, ef, dst2d, wfij_t, a1, p36, bm=512):
    e_pad = g_ni.shape[0]
    grid = e_pad // bm

    def body(ni_ref, nj_ref, ef_ref, dst_ref, w_ref, a_ref, p_ref,
             ee_ref, idx_ref, hs_ref):
        i = pl.program_id(0)
        f = (ni_ref[...] + nj_ref[...]
             + jnp.dot(ef_ref[...], w_ref[...], preferred_element_type=f32))
        t = jnp.where(f > 0, f, 0.01 * f)
        e = jnp.dot(t, a_ref[...], preferred_element_type=f32)
        rows = i * bm + lax.broadcasted_iota(i32, (bm, H), 0)
        ee = jnp.where(rows < E_GG, jnp.exp(e), 0.0)
        dst = dst_ref[...]
        group = lax.broadcasted_iota(i32, (bm, 128), 1) // 8
        ee_ref[...] = jnp.tile(ee, (1, 16)) * (group == (dst % 16)).astype(f32)
        idx_ref[...] = dst // 16
        hs_ref[...] = jnp.dot(t, p_ref[...], preferred_element_type=f32)

    return pl.pallas_call(
        body, grid=(grid,),
        in_specs=[
            pl.BlockSpec((bm, 384), lambda i: (i, 0)),
            pl.BlockSpec((bm, 384), lambda i: (i, 0)),
            pl.BlockSpec((bm, 16), lambda i: (i, 0)),
            pl.BlockSpec((bm, 1), lambda i: (i, 0)),
            pl.BlockSpec((16, 384), lambda i: (0, 0)),
            pl.BlockSpec((384, H), lambda i: (0, 0)),
            pl.BlockSpec((384, 36), lambda i: (0, 0)),
        ],
        out_specs=[pl.BlockSpec((bm, 128), lambda i: (i, 0)),
                   pl.BlockSpec((bm, 1), lambda i: (i, 0)),
                   pl.BlockSpec((bm, 36), lambda i: (i, 0))],
        out_shape=[jax.ShapeDtypeStruct((e_pad, 128), f32),
                   jax.ShapeDtypeStruct((e_pad, 1), i32),
                   jax.ShapeDtypeStruct((e_pad, 36), f32)],
    )(g_ni, g_nj, ef, dst2d, wfij_t, a1, p36)


def _tc_combine_pad(den_p, n_seg, bm=512):
    grid = n_seg // bm
    nb = n_seg // bm

    def body(p0_ref, p1_ref, out_ref):
        s = p0_ref[...] + p1_ref[...] + 1e-30
        out_ref[...] = jnp.concatenate([s, jnp.zeros((bm, 120), f32)], axis=1)

    return pl.pallas_call(
        body, grid=(grid,),
        in_specs=[pl.BlockSpec((bm, H), lambda i: (i, 0)),
                  pl.BlockSpec((bm, H), lambda i: (i + nb, 0))],
        out_specs=pl.BlockSpec((bm, 128), lambda i: (i, 0)),
        out_shape=jax.ShapeDtypeStruct((n_seg, 128), f32),
    )(den_p, den_p)


def _tc_edge_lg(lg_ef, y2c0, y2c1, g_ni, g_nj, dst2d, wfij_t, a2, bm=512):
    e_pad = lg_ef.shape[0]
    grid = e_pad // bm
    hb = bm // 2

    def body(ef_ref, p00, p01, p10, p11, ni_ref, nj_ref, dst_ref,
             w_ref, a_ref, ee_ref, idx_ref):
        i = pl.program_id(0)
        y2 = jnp.concatenate([p00[...] + p01[...], p10[...] + p11[...]], axis=1)
        brep = jnp.repeat(y2, 2, axis=0)
        f = (ni_ref[...] + nj_ref[...]
             + jnp.dot(brep + ef_ref[...], w_ref[...], preferred_element_type=f32))
        t = jnp.where(f > 0, f, 0.01 * f)
        e = jnp.dot(t, a_ref[...], preferred_element_type=f32)
        rows = i * bm + lax.broadcasted_iota(i32, (bm, H), 0)
        ee = jnp.where(rows < E_LG, jnp.exp(e), 0.0)
        dst = dst_ref[...]
        group = lax.broadcasted_iota(i32, (bm, 128), 1) // 8
        ee_ref[...] = jnp.tile(ee, (1, 16)) * (group == (dst % 16)).astype(f32)
        idx_ref[...] = dst // 16

    ng_blocks = NG_PAD // hb
    return pl.pallas_call(
        body, grid=(grid,),
        in_specs=[
            pl.BlockSpec((bm, 128), lambda i: (i, 0)),
            pl.BlockSpec((hb, 64), lambda i: (i, 0)),
            pl.BlockSpec((hb, 64), lambda i: (i + ng_blocks, 0)),
            pl.BlockSpec((hb, 64), lambda i: (i, 0)),
            pl.BlockSpec((hb, 64), lambda i: (i + ng_blocks, 0)),
            pl.BlockSpec((bm, 1024), lambda i: (i, 0)),
            pl.BlockSpec((bm, 1024), lambda i: (i, 0)),
            pl.BlockSpec((bm, 1), lambda i: (i, 0)),
            pl.BlockSpec((128, 1024), lambda i: (0, 0)),
            pl.BlockSpec((1024, H), lambda i: (0, 0)),
        ],
        out_specs=[pl.BlockSpec((bm, 128), lambda i: (i, 0)),
                   pl.BlockSpec((bm, 1), lambda i: (i, 0))],
        out_shape=[jax.ShapeDtypeStruct((e_pad, 128), f32),
                   jax.ShapeDtypeStruct((e_pad, 1), i32)],
    )(lg_ef, y2c0, y2c0, y2c1, y2c1, g_ni, g_nj, dst2d, wfij_t, a2)


def _tc_softmax_div(ee_spread, g_den, dst2d, bm=1024):
    e_pad = ee_spread.shape[0]
    grid = e_pad // bm

    def body(ee_ref, d_ref, dst_ref, al_ref, ar_ref):
        ee = jnp.sum(ee_ref[...].reshape(bm, 16, H), axis=1)
        a = ee / d_ref[...][:, :H]
        par = (dst_ref[...] % 2).astype(f32)
        al_ref[...] = a * (1.0 - par)
        ar_ref[...] = a * par

    return pl.pallas_call(
        body, grid=(grid,),
        in_specs=[pl.BlockSpec((bm, 128), lambda i: (i, 0)),
                  pl.BlockSpec((bm, 128), lambda i: (i, 0)),
                  pl.BlockSpec((bm, 1), lambda i: (i, 0))],
        out_specs=[pl.BlockSpec((bm, H), lambda i: (i, 0)),
                   pl.BlockSpec((bm, H), lambda i: (i, 0))],
        out_shape=[jax.ShapeDtypeStruct((e_pad, H), f32),
                   jax.ShapeDtypeStruct((e_pad, H), f32)],
    )(ee_spread, g_den, dst2d)


def _tc_softmax_div_spread(ee_spread, g_den, src2d, bm=1024):
    e_pad = ee_spread.shape[0]
    grid = e_pad // bm

    def body(ee_ref, d_ref, src_ref, a_ref, idx_ref):
        ee = jnp.sum(ee_ref[...].reshape(bm, 16, H), axis=1)
        a = ee / d_ref[...][:, :H]
        src = src_ref[...]
        group = lax.broadcasted_iota(i32, (bm, 128), 1) // 8
        a_ref[...] = jnp.tile(a, (1, 16)) * (group == (src % 16)).astype(f32)
        idx_ref[...] = src // 16

    return pl.pallas_call(
        body, grid=(grid,),
        in_specs=[pl.BlockSpec((bm, 128), lambda i: (i, 0)),
                  pl.BlockSpec((bm, 128), lambda i: (i, 0)),
                  pl.BlockSpec((bm, 1), lambda i: (i, 0))],
        out_specs=[pl.BlockSpec((bm, 128), lambda i: (i, 0)),
                   pl.BlockSpec((bm, 1), lambda i: (i, 0))],
        out_shape=[jax.ShapeDtypeStruct((e_pad, 128), f32),
                   jax.ShapeDtypeStruct((e_pad, 1), i32)],
    )(ee_spread, g_den, src2d)


def _tc_final(bp, h2, hs36, ro_w1t, ro_b1, ro_w2t, ro_b2):
    grid = 20
    bn = NL_PAD // grid
    bh = N_GG // grid

    def body(b0_ref, b1_ref, h2_ref, hs_ref, w1_ref, b1v_ref, w2_ref, b2v_ref,
             out_ref, s1, s2):
        i = pl.program_id(0)

        @pl.when(i == 0)
        def _():
            s1[...] = jnp.zeros((1, 128), f32)
            s2[...] = jnp.zeros((1, 36), f32)

        b = b0_ref[...] + b1_ref[...]
        h2b = h2_ref[...].reshape(bn, H, 128)
        contrib = jnp.sum(b[:, :, None] * h2b, axis=(0, 1))
        s1[...] = s1[...] + contrib.reshape(1, 128)
        s2[...] = s2[...] + jnp.sum(hs_ref[...], axis=0).reshape(1, 36)

        @pl.when(i == grid - 1)
        def _():
            y1 = s1[0, :] * (1.0 / N_LG)
            y2p = s2[0, :] * (1.0 / N_GG)
            y = jnp.concatenate([y1, y2p])
            x = jax.nn.sigmoid(jnp.dot(y, w1_ref[...],
                                       preferred_element_type=f32) + b1v_ref[0, :])
            o = jax.nn.sigmoid(jnp.dot(x, w2_ref[...],
                                       preferred_element_type=f32)[0] + b2v_ref[0, 0])
            out_ref[...] = jnp.full((8, 128), o, f32)

    nl_blocks = NL_PAD // bn
    return pl.pallas_call(
        body, grid=(grid,),
        in_specs=[
            pl.BlockSpec((bn, H), lambda i: (i, 0)),
            pl.BlockSpec((bn, H), lambda i: (i + nl_blocks, 0)),
            pl.BlockSpec((bn, 1024), lambda i: (i, 0)),
            pl.BlockSpec((bh, 36), lambda i: (i, 0)),
            pl.BlockSpec((164, 128), lambda i: (0, 0)),
            pl.BlockSpec((1, 128), lambda i: (0, 0)),
            pl.BlockSpec((128, 1), lambda i: (0, 0)),
            pl.BlockSpec((1, 1), lambda i: (0, 0)),
        ],
        out_specs=pl.BlockSpec((8, 128), lambda i: (0, 0)),
        out_shape=jax.ShapeDtypeStruct((8, 128), f32),
        scratch_shapes=[pltpu.VMEM((1, 128), f32), pltpu.VMEM((1, 36), f32)],
    )(bp, bp, h2, hs36, ro_w1t, ro_b1, ro_w2t, ro_b2)


_PERM0 = np.add.outer(np.arange(H) * 128, np.arange(64)).reshape(-1)
_PERM1 = _PERM0 + 64
_P36_PAD = np.concatenate([np.tile(np.eye(36, dtype=np.float32), (H, 1)),
                           np.zeros((96, 36), np.float32)], axis=0)
_A1_ROWS = np.arange(288)
_A1_COLS = np.repeat(np.arange(H), 36)
_A2_ROWS = np.arange(1024)
_A2_COLS = np.repeat(np.arange(H), 128)


def kernel(gg_nfeat, gg_efeat, gg_edge_index, lg_nfeat, lg_efeat, lg_edge_index,
           l1_Wn, l1_bn, l1_Wni, l1_Wfij, l1_Wnj, l1_attn, l1_bias,
           l2_Wn, l2_bn, l2_Wni, l2_Wfij, l2_Wnj, l2_attn, l2_bias,
           ro_W1, ro_b1, ro_W2, ro_b2):
    src_g = jnp.concatenate([gg_edge_index[0], jnp.zeros(EG_PAD - E_GG, i32)])
    dst_g = jnp.concatenate([gg_edge_index[1], jnp.zeros(EG_PAD - E_GG, i32)])
    src_l = jnp.concatenate([lg_edge_index[0], jnp.zeros(EL_PAD - E_LG, i32)])
    dst_l = jnp.concatenate([lg_edge_index[1], jnp.zeros(EL_PAD - E_LG, i32)])
    gg_nf = jnp.concatenate([gg_nfeat, jnp.zeros((NG_PAD - N_GG, 128), f32)])
    lg_nf = jnp.concatenate([lg_nfeat, jnp.zeros((NL_PAD - N_LG, 128), f32)])
    gg_ef = jnp.concatenate([gg_efeat, jnp.zeros((EG_PAD - E_GG, 16), f32)])
    lg_ef = jnp.concatenate([lg_efeat, jnp.zeros((EL_PAD - E_LG, 128), f32)])

    wn1c0_t = l1_Wn[_PERM0, :].T
    wn1c1_t = l1_Wn[_PERM1, :].T
    bn1c0 = l1_bn[_PERM0].reshape(1, 512)
    bn1c1 = l1_bn[_PERM1].reshape(1, 512)
    a1 = jnp.zeros((384, H), f32).at[_A1_ROWS, _A1_COLS].set(l1_attn.reshape(-1))
    a2 = jnp.zeros((1024, H), f32).at[_A2_ROWS, _A2_COLS].set(l2_attn.reshape(-1))
    zero96 = jnp.zeros((1, 96), f32)
    zero1024 = jnp.zeros((1, 1024), f32)
    wni1_t = jnp.concatenate([l1_Wni.T, jnp.zeros((128, 96), f32)], axis=1)
    wnj1_t = jnp.concatenate([l1_Wnj.T, jnp.zeros((128, 96), f32)], axis=1)
    wfij1_t = jnp.concatenate([l1_Wfij.T, jnp.zeros((16, 96), f32)], axis=1)
    bias1 = jnp.concatenate([l1_bias.reshape(1, 288), zero96], axis=1)

    f_ni1, f_nj1, h1c0, h1c1 = _tc_matmul_multi(
        gg_nf,
        [(wni1_t, bias1),
         (wnj1_t, jnp.zeros((1, 384), f32)),
         (wn1c0_t, bn1c0),
         (wn1c1_t, bn1c1)],
        bm=512)

    dst_g2d = dst_g.reshape(-1, 1)
    g_ni1, g_nj1 = _sc_gather2(f_ni1, f_nj1, src_g, dst_g, 64)
    ee1s, idx16_g, hs36 = _tc_edge_gg(
        g_ni1, g_nj1, gg_ef, dst_g2d, wfij1_t, a1, jnp.asarray(_P36_PAD))

    n16g = NG_PAD // 16
    den1p = _sc_scatter_add(ee1s, idx16_g.reshape(-1), n16g, 128)
    den1_8 = jnp.concatenate([den1p[:n16g].reshape(NG_PAD, H),
                              den1p[n16g:].reshape(NG_PAD, H)], axis=0)
    den1_pad = _tc_combine_pad(den1_8, NG_PAD)
    g_den1 = _sc_gather(den1_pad, dst_g, 128)
    afl1, afr1 = _tc_softmax_div(ee1s, g_den1, dst_g2d)

    nhg = NG_PAD // 2
    y2c0p = _sc_wsum(h1c0, afl1.reshape(-1), afr1.reshape(-1),
                     src_g, dst_g, nhg, 32)
    y2c1p = _sc_wsum(h1c1, afl1.reshape(-1), afr1.reshape(-1),
                     src_g, dst_g, nhg, 32)
    y2c0 = jnp.concatenate([y2c0p[:nhg].reshape(NG_PAD, 64),
                            y2c0p[nhg:].reshape(NG_PAD, 64)], axis=0)
    y2c1 = jnp.concatenate([y2c1p[:nhg].reshape(NG_PAD, 64),
                            y2c1p[nhg:].reshape(NG_PAD, 64)], axis=0)

    f_ni2, f_nj2, h2 = _tc_matmul_multi(
        lg_nf,
        [(l2_Wni.T, l2_bias.reshape(1, 1024)),
         (l2_Wnj.T, zero1024),
         (l2_Wn.T, l2_bn.reshape(1, 1024))],
        bm=512)

    dst_l2d = dst_l.reshape(-1, 1)
    src_l2d = src_l.reshape(-1, 1)
    g_ni2 = _sc_gather(f_ni2, src_l, 32)
    g_nj2 = _sc_gather(f_nj2, dst_l, 32)
    ee2s, idx16_l = _tc_edge_lg(lg_ef, y2c0, y2c1, g_ni2, g_nj2, dst_l2d,
                                l2_Wfij.T, a2)

    n16l = NL_PAD // 16
    den2p = _sc_scatter_add(ee2s, idx16_l.reshape(-1), n16l, 128)
    den2_8 = jnp.concatenate([den2p[:n16l].reshape(NL_PAD, H),
                              den2p[n16l:].reshape(NL_PAD, H)], axis=0)
    den2_pad = _tc_combine_pad(den2_8, NL_PAD)
    g_den2 = _sc_gather(den2_pad, dst_l, 128)
    a2s, idx16_s = _tc_softmax_div_spread(ee2s, g_den2, src_l2d)

    bp128 = _sc_scatter_add(a2s, idx16_s.reshape(-1), n16l, 128)
    bp = jnp.concatenate([bp128[:n16l].reshape(NL_PAD, H),
                          bp128[n16l:].reshape(NL_PAD, H)], axis=0)

    out_pad = _tc_final(bp, h2, hs36, ro_W1.T, ro_b1.reshape(1, 128),
                        ro_W2.T, ro_b2.reshape(1, 1))
    return out_pad[0, 0:1]


# speedup vs baseline: 5.8846x; 1.0741x over previous
import functools

import numpy as np
import jax
import jax.numpy as jnp
from jax import lax
from jax.experimental import pallas as pl
from jax.experimental.pallas import tpu as pltpu
from jax.experimental.pallas import tpu_sc as plsc

H = 8
N_GG, E_GG, N_LG, E_LG = 20000, 160000, 10000, 40000
NG_PAD, EG_PAD = 20480, 163840
NL_PAD, EL_PAD = 10240, 40960
NC, NS = 2, 16
NW = NC * NS

f32 = jnp.float32
i32 = jnp.int32


def _sc_mesh():
    return plsc.VectorSubcoreMesh(
        core_axis_name="c", subcore_axis_name="s", num_cores=NC, num_subcores=NS)


def _sc_gather(table, idx, chunk):
    N, D = table.shape
    E = idx.shape[0]
    per_w = E // NW
    n_ch = per_w // chunk
    assert per_w % chunk == 0 and chunk % 8 == 0 and chunk <= 128
    assert n_ch % 2 == 0

    @functools.partial(
        pl.kernel,
        out_type=jax.ShapeDtypeStruct((E, D), f32),
        mesh=_sc_mesh(),
        scratch_types=[
            pltpu.VMEM((2, chunk), i32),
            pltpu.VMEM((2, chunk, D), f32),
            pltpu.SemaphoreType.DMA,
            pltpu.SemaphoreType.DMA,
            pltpu.SemaphoreType.DMA,
            pltpu.SemaphoreType.DMA,
        ],
    )
    def k(table_hbm, idx_hbm, out_hbm, idx_v, rows_v, g0, g1, o0, o1):
        wid = lax.axis_index("s") * NC + lax.axis_index("c")
        base0 = wid * per_w
        gs = (g0, g1)
        os_ = (o0, o1)

        def fire_g(ci, b):
            pltpu.sync_copy(idx_hbm.at[pl.ds(base0 + ci * chunk, chunk)],
                            idx_v.at[b])
            pltpu.async_copy(table_hbm.at[idx_v.at[b]], rows_v.at[b], gs[b])

        def wait_g(ci, b):
            pltpu.make_async_copy(table_hbm.at[idx_v.at[b]],
                                  rows_v.at[b], gs[b]).wait()

        def fire_o(ci, b):
            pltpu.async_copy(rows_v.at[b],
                             out_hbm.at[pl.ds(base0 + ci * chunk, chunk)],
                             os_[b])

        def wait_o(ci, b):
            pltpu.make_async_copy(rows_v.at[b],
                                  out_hbm.at[pl.ds(base0 + ci * chunk, chunk)],
                                  os_[b]).wait()

        fire_g(0, 0)

        def body(ci2, carry):
            for b in range(2):
                ci = ci2 * 2 + b
                nb = 1 - b

                @pl.when(ci >= 1)
                def _():
                    wait_o(ci - 1, nb)

                @pl.when(ci + 1 < n_ch)
                def _():
                    fire_g(ci + 1, nb)

                wait_g(ci, b)
                fire_o(ci, b)
            return carry

        lax.fori_loop(0, n_ch // 2, body, 0)
        wait_o(n_ch - 1, 1)

    return k(table, idx)


def _sc_gather2(table_a, table_b, idx_a, idx_b, chunk):
    N, D = table_a.shape
    E = idx_a.shape[0]
    per_w = E // NW
    n_ch = per_w // chunk
    assert per_w % chunk == 0 and chunk % 8 == 0 and chunk <= 128
    assert n_ch % 2 == 0

    @functools.partial(
        pl.kernel,
        out_type=[jax.ShapeDtypeStruct((E, D), f32),
                  jax.ShapeDtypeStruct((E, D), f32)],
        mesh=_sc_mesh(),
        scratch_types=[
            pltpu.VMEM((2, chunk), i32),
            pltpu.VMEM((2, chunk), i32),
            pltpu.VMEM((2, chunk, D), f32),
            pltpu.VMEM((2, chunk, D), f32),
            pltpu.SemaphoreType.DMA,
            pltpu.SemaphoreType.DMA,
            pltpu.SemaphoreType.DMA,
            pltpu.SemaphoreType.DMA,
            pltpu.SemaphoreType.DMA,
            pltpu.SemaphoreType.DMA,
            pltpu.SemaphoreType.DMA,
            pltpu.SemaphoreType.DMA,
        ],
    )
    def k(ta_hbm, tb_hbm, ia_hbm, ib_hbm, oa_hbm, ob_hbm,
          ia_v, ib_v, ra_v, rb_v, ga0, ga1, gb0, gb1, oa0, oa1, ob0, ob1):
        wid = lax.axis_index("s") * NC + lax.axis_index("c")
        base0 = wid * per_w
        ga = (ga0, ga1)
        gb = (gb0, gb1)
        oa = (oa0, oa1)
        ob = (ob0, ob1)

        def fire_g(ci, b):
            sl = pl.ds(base0 + ci * chunk, chunk)
            pltpu.sync_copy(ia_hbm.at[sl], ia_v.at[b])
            pltpu.sync_copy(ib_hbm.at[sl], ib_v.at[b])
            pltpu.async_copy(ta_hbm.at[ia_v.at[b]], ra_v.at[b], ga[b])
            pltpu.async_copy(tb_hbm.at[ib_v.at[b]], rb_v.at[b], gb[b])

        def wait_g(ci, b):
            pltpu.make_async_copy(ta_hbm.at[ia_v.at[b]], ra_v.at[b],
                                  ga[b]).wait()
            pltpu.make_async_copy(tb_hbm.at[ib_v.at[b]], rb_v.at[b],
                                  gb[b]).wait()

        def fire_o(ci, b):
            sl = pl.ds(base0 + ci * chunk, chunk)
            pltpu.async_copy(ra_v.at[b], oa_hbm.at[sl], oa[b])
            pltpu.async_copy(rb_v.at[b], ob_hbm.at[sl], ob[b])

        def wait_o(ci, b):
            sl = pl.ds(base0 + ci * chunk, chunk)
            pltpu.make_async_copy(ra_v.at[b], oa_hbm.at[sl], oa[b]).wait()
            pltpu.make_async_copy(rb_v.at[b], ob_hbm.at[sl], ob[b]).wait()

        fire_g(0, 0)

        def body(ci2, carry):
            for b in range(2):
                ci = ci2 * 2 + b
                nb = 1 - b

                @pl.when(ci >= 1)
                def _():
                    wait_o(ci - 1, nb)

                @pl.when(ci + 1 < n_ch)
                def _():
                    fire_g(ci + 1, nb)

                wait_g(ci, b)
                fire_o(ci, b)
            return carry

        lax.fori_loop(0, n_ch // 2, body, 0)
        wait_o(n_ch - 1, 1)

    return k(table_a, table_b, idx_a, idx_b)


def _sc_scatter_add(vals, idx, n_seg, chunk):
    E, D = vals.shape
    per_w = E // NW
    n_ch = per_w // chunk
    rpt = n_seg // NS
    assert per_w % chunk == 0 and n_seg % NS == 0
    zeros = jnp.zeros((rpt, D), f32)

    assert n_ch % 2 == 0

    @functools.partial(
        pl.kernel,
        out_type=jax.ShapeDtypeStruct((NC * n_seg, D), f32),
        mesh=_sc_mesh(),
        scratch_types=[
            pltpu.VMEM((2, chunk), i32),
            pltpu.VMEM((2, chunk, D), f32),
            pltpu.VMEM_SHARED((n_seg, D), f32),
            pltpu.SemaphoreType.DMA,
            pltpu.SemaphoreType.DMA,
            pltpu.SemaphoreType.DMA,
            pltpu.SemaphoreType.DMA,
        ],
    )
    def k(vals_hbm, idx_hbm, z_hbm, out_hbm, idx_v, vals_v, acc,
          v0, v1, w0, w1):
        cid = lax.axis_index("c")
        sid = lax.axis_index("s")
        wid = sid * NC + cid
        vs = (v0, v1)
        ws = (w0, w1)

        def fire_v(ci, b):
            base = wid * per_w + ci * chunk
            pltpu.sync_copy(idx_hbm.at[pl.ds(base, chunk)], idx_v.at[b])
            pltpu.async_copy(vals_hbm.at[pl.ds(base, chunk)], vals_v.at[b],
                             vs[b])

        def wait_v(ci, b):
            base = wid * per_w + ci * chunk
            pltpu.make_async_copy(vals_hbm.at[pl.ds(base, chunk)],
                                  vals_v.at[b], vs[b]).wait()

        def fire_s(ci, b):
            pltpu.async_copy(vals_v.at[b], acc.at[idx_v.at[b]], ws[b],
                             add=True)

        def wait_s(ci, b):
            pltpu.make_async_copy(vals_v.at[b], acc.at[idx_v.at[b]],
                                  ws[b]).wait()

        pltpu.sync_copy(z_hbm, acc.at[pl.ds(sid * rpt, rpt)])
        plsc.subcore_barrier()
        fire_v(0, 0)

        def body(ci2, carry):
            for b in range(2):
                ci = ci2 * 2 + b
                nb = 1 - b

                @pl.when(ci >= 1)
                def _():
                    wait_s(ci - 1, nb)

                @pl.when(ci + 1 < n_ch)
                def _():
                    fire_v(ci + 1, nb)

                wait_v(ci, b)
                fire_s(ci, b)
            return carry

        lax.fori_loop(0, n_ch // 2, body, 0)
        wait_s(n_ch - 1, 1)
        plsc.subcore_barrier()
        pltpu.sync_copy(acc.at[pl.ds(sid * rpt, rpt)],
                        out_hbm.at[pl.ds(cid * n_seg + sid * rpt, rpt)])

    return k(vals, idx, zeros)


def _sc_wsum(table_half, af_l, af_r, src, dst, n_half, chunk):
    E = src.shape[0]
    per_w = E // NW
    n_ch = per_w // chunk
    rpt = n_half // NS
    zeros = jnp.zeros((rpt, 128), f32)

    assert n_ch % 2 == 0

    @functools.partial(
        pl.kernel,
        out_type=jax.ShapeDtypeStruct((NC * n_half, 128), f32),
        mesh=_sc_mesh(),
        scratch_types=[
            pltpu.VMEM((2, chunk), i32),
            pltpu.VMEM((2, chunk), i32),
            pltpu.VMEM((2, chunk), i32),
            pltpu.VMEM((2, chunk * 8), f32),
            pltpu.VMEM((2, chunk * 8), f32),
            pltpu.VMEM((2, chunk, 512), f32),
            pltpu.VMEM((chunk, 128), f32),
            pltpu.VMEM_SHARED((n_half, 128), f32),
            pltpu.SemaphoreType.DMA,
            pltpu.SemaphoreType.DMA,
            pltpu.SemaphoreType.DMA,
            pltpu.SemaphoreType.DMA,
        ],
    )
    def k(tab_hbm, afl_hbm, afr_hbm, src_hbm, dst_hbm, z_hbm, out_hbm,
          src_v, dst_v, idx2_v, afl_v, afr_v, rows_v, msg_v, acc,
          g0, g1, x0, x1):
        cid = lax.axis_index("c")
        sid = lax.axis_index("s")
        wid = sid * NC + cid
        gs = (g0, g1)
        xs = (x0, x1)

        def fire(ci, b):
            base = wid * per_w + ci * chunk
            pltpu.async_copy(dst_hbm.at[pl.ds(base, chunk)], dst_v.at[b], xs[b])
            pltpu.async_copy(afl_hbm.at[pl.ds(base * 8, chunk * 8)],
                             afl_v.at[b], xs[b])
            pltpu.async_copy(afr_hbm.at[pl.ds(base * 8, chunk * 8)],
                             afr_v.at[b], xs[b])
            pltpu.sync_copy(src_hbm.at[pl.ds(base, chunk)], src_v.at[b])
            pltpu.async_copy(tab_hbm.at[src_v.at[b]], rows_v.at[b], gs[b])

        def wait(ci, b):
            base = wid * per_w + ci * chunk
            pltpu.make_async_copy(tab_hbm.at[src_v.at[b]], rows_v.at[b],
                                  gs[b]).wait()
            pltpu.make_async_copy(dst_hbm.at[pl.ds(base, chunk)], dst_v.at[b],
                                  xs[b]).wait()
            pltpu.make_async_copy(afl_hbm.at[pl.ds(base * 8, chunk * 8)],
                                  afl_v.at[b], xs[b]).wait()
            pltpu.make_async_copy(afr_hbm.at[pl.ds(base * 8, chunk * 8)],
                                  afr_v.at[b], xs[b]).wait()

        pltpu.sync_copy(z_hbm, acc.at[pl.ds(sid * rpt, rpt)])
        plsc.subcore_barrier()
        fire(0, 0)

        def body(ci2, carry):
            for b in range(2):
                ci = ci2 * 2 + b

                @pl.when(ci + 1 < n_ch)
                def _():
                    fire(ci + 1, 1 - b)

                wait(ci, b)

                def halve(g, c2):
                    d16 = dst_v[b, pl.ds(g * 16, 16)]
                    idx2_v[b, pl.ds(g * 16, 16)] = lax.shift_right_logical(d16, 1)
                    return c2

                lax.fori_loop(0, chunk // 16, halve, 0)

                def edge_pair(j, c2):
                    avl = afl_v[b, pl.ds(j * 16, 16)]
                    avr = afr_v[b, pl.ds(j * 16, 16)]
                    for r in range(2):
                        i = j * 2 + r
                        accl = [jnp.zeros((16,), f32) for _ in range(4)]
                        accr = [jnp.zeros((16,), f32) for _ in range(4)]
                        for h in range(8):
                            sl = avl[r * 8 + h]
                            sr = avr[r * 8 + h]
                            for q in range(4):
                                row = rows_v[b, i, pl.ds(h * 64 + q * 16, 16)]
                                accl[q] = accl[q] + sl * row
                                accr[q] = accr[q] + sr * row
                        for q in range(4):
                            msg_v[i, pl.ds(q * 16, 16)] = accl[q]
                            msg_v[i, pl.ds(64 + q * 16, 16)] = accr[q]
                    return c2

                lax.fori_loop(0, chunk // 2, edge_pair, 0)
                pltpu.sync_copy(msg_v, acc.at[idx2_v.at[b]], add=True)
            return carry

        lax.fori_loop(0, n_ch // 2, body, 0)
        plsc.subcore_barrier()
        pltpu.sync_copy(acc.at[pl.ds(sid * rpt, rpt)],
                        out_hbm.at[pl.ds(cid * n_half + sid * rpt, rpt)])

    return k(table_half, af_l, af_r, src, dst, zeros)


def _tc_matmul_multi(x, wbs, bm):
    m, kdim = x.shape
    grid = m // bm
    n = len(wbs)

    def body(*refs):
        xb = refs[0][...]
        for j in range(n):
            w = refs[1 + 2 * j][...]
            b = refs[2 + 2 * j][...]
            refs[1 + 2 * n + j][...] = (
                jnp.dot(xb, w, preferred_element_type=f32) + b)

    in_specs = [pl.BlockSpec((bm, kdim), lambda i: (i, 0))]
    ins = [x]
    for (w, b) in wbs:
        in_specs.append(pl.BlockSpec(w.shape, lambda i: (0, 0)))
        in_specs.append(pl.BlockSpec(b.shape, lambda i: (0, 0)))
        ins.extend([w, b])
    out_shape = [jax.ShapeDtypeStruct((m, w.shape[1]), f32) for (w, _) in wbs]
    out_specs = [pl.BlockSpec((bm, w.shape[1]), lambda i: (i, 0)) for (w, _) in wbs]
    return pl.pallas_call(body, grid=(grid,), in_specs=in_specs,
                          out_specs=out_specs, out_shape=out_shape)(*ins)


def _tc_edge_gg(g_ni, g_nj, ef, dst2d, wfij_t, a1, p36, bm=512):
    e_pad = g_ni.shape[0]
    grid = e_pad // bm

    def body(ni_ref, nj_ref, ef_ref, dst_ref, w_ref, a_ref, p_ref,
             ee_ref, idx_ref, hs_ref):
        i = pl.program_id(0)
        f = (ni_ref[...] + nj_ref[...]
             + jnp.dot(ef_ref[...], w_ref[...], preferred_element_type=f32))
        t = jnp.where(f > 0, f, 0.01 * f)
        e = jnp.dot(t, a_ref[...], preferred_element_type=f32)
        rows = i * bm + lax.broadcasted_iota(i32, (bm, H), 0)
        ee = jnp.where(rows < E_GG, jnp.exp(e), 0.0)
        dst = dst_ref[...]
        group = lax.broadcasted_iota(i32, (bm, 128), 1) // 8
        ee_ref[...] = jnp.tile(ee, (1, 16)) * (group == (dst % 16)).astype(f32)
        idx_ref[...] = dst // 16
        hs_ref[...] = jnp.dot(t, p_ref[...], preferred_element_type=f32)

    return pl.pallas_call(
        body, grid=(grid,),
        in_specs=[
            pl.BlockSpec((bm, 384), lambda i: (i, 0)),
            pl.BlockSpec((bm, 384), lambda i: (i, 0)),
            pl.BlockSpec((bm, 16), lambda i: (i, 0)),
            pl.BlockSpec((bm, 1), lambda i: (i, 0)),
            pl.BlockSpec((16, 384), lambda i: (0, 0)),
            pl.BlockSpec((384, H), lambda i: (0, 0)),
            pl.BlockSpec((384, 36), lambda i: (0, 0)),
        ],
        out_specs=[pl.BlockSpec((bm, 128), lambda i: (i, 0)),
                   pl.BlockSpec((bm, 1), lambda i: (i, 0)),
                   pl.BlockSpec((bm, 36), lambda i: (i, 0))],
        out_shape=[jax.ShapeDtypeStruct((e_pad, 128), f32),
                   jax.ShapeDtypeStruct((e_pad, 1), i32),
                   jax.ShapeDtypeStruct((e_pad, 36), f32)],
    )(g_ni, g_nj, ef, dst2d, wfij_t, a1, p36)


def _tc_combine_pad(den_p, n_seg, bm=512):
    grid = n_seg // bm
    nb = n_seg // bm

    def body(p0_ref, p1_ref, out_ref):
        s = p0_ref[...] + p1_ref[...] + 1e-30
        out_ref[...] = jnp.concatenate([s, jnp.zeros((bm, 120), f32)], axis=1)

    return pl.pallas_call(
        body, grid=(grid,),
        in_specs=[pl.BlockSpec((bm, H), lambda i: (i, 0)),
                  pl.BlockSpec((bm, H), lambda i: (i + nb, 0))],
        out_specs=pl.BlockSpec((bm, 128), lambda i: (i, 0)),
        out_shape=jax.ShapeDtypeStruct((n_seg, 128), f32),
    )(den_p, den_p)


def _tc_edge_lg(lg_ef, y2c0, y2c1, g_ni, g_nj, dst2d, wfij_t, a2, bm=512):
    e_pad = lg_ef.shape[0]
    grid = e_pad // bm
    hb = bm // 2

    def body(ef_ref, p00, p01, p10, p11, ni_ref, nj_ref, dst_ref,
             w_ref, a_ref, ee_ref, idx_ref):
        i = pl.program_id(0)
        y2 = jnp.concatenate([p00[...] + p01[...], p10[...] + p11[...]], axis=1)
        brep = jnp.repeat(y2, 2, axis=0)
        f = (ni_ref[...] + nj_ref[...]
             + jnp.dot(brep + ef_ref[...], w_ref[...], preferred_element_type=f32))
        t = jnp.where(f > 0, f, 0.01 * f)
        e = jnp.dot(t, a_ref[...], preferred_element_type=f32)
        rows = i * bm + lax.broadcasted_iota(i32, (bm, H), 0)
        ee = jnp.where(rows < E_LG, jnp.exp(e), 0.0)
        dst = dst_ref[...]
        group = lax.broadcasted_iota(i32, (bm, 128), 1) // 8
        ee_ref[...] = jnp.tile(ee, (1, 16)) * (group == (dst % 16)).astype(f32)
        idx_ref[...] = dst // 16

    ng_blocks = NG_PAD // hb
    return pl.pallas_call(
        body, grid=(grid,),
        in_specs=[
            pl.BlockSpec((bm, 128), lambda i: (i, 0)),
            pl.BlockSpec((hb, 64), lambda i: (i, 0)),
            pl.BlockSpec((hb, 64), lambda i: (i + ng_blocks, 0)),
            pl.BlockSpec((hb, 64), lambda i: (i, 0)),
            pl.BlockSpec((hb, 64), lambda i: (i + ng_blocks, 0)),
            pl.BlockSpec((bm, 1024), lambda i: (i, 0)),
            pl.BlockSpec((bm, 1024), lambda i: (i, 0)),
            pl.BlockSpec((bm, 1), lambda i: (i, 0)),
            pl.BlockSpec((128, 1024), lambda i: (0, 0)),
            pl.BlockSpec((1024, H), lambda i: (0, 0)),
        ],
        out_specs=[pl.BlockSpec((bm, 128), lambda i: (i, 0)),
                   pl.BlockSpec((bm, 1), lambda i: (i, 0))],
        out_shape=[jax.ShapeDtypeStruct((e_pad, 128), f32),
                   jax.ShapeDtypeStruct((e_pad, 1), i32)],
    )(lg_ef, y2c0, y2c0, y2c1, y2c1, g_ni, g_nj, dst2d, wfij_t, a2)


def _tc_softmax_div(ee_spread, g_den, dst2d, bm=1024):
    e_pad = ee_spread.shape[0]
    grid = e_pad // bm

    def body(ee_ref, d_ref, dst_ref, al_ref, ar_ref):
        ee = jnp.sum(ee_ref[...].reshape(bm, 16, H), axis=1)
        a = ee / d_ref[...][:, :H]
        par = (dst_ref[...] % 2).astype(f32)
        al_ref[...] = a * (1.0 - par)
        ar_ref[...] = a * par

    return pl.pallas_call(
        body, grid=(grid,),
        in_specs=[pl.BlockSpec((bm, 128), lambda i: (i, 0)),
                  pl.BlockSpec((bm, 128), lambda i: (i, 0)),
                  pl.BlockSpec((bm, 1), lambda i: (i, 0))],
        out_specs=[pl.BlockSpec((bm, H), lambda i: (i, 0)),
                   pl.BlockSpec((bm, H), lambda i: (i, 0))],
        out_shape=[jax.ShapeDtypeStruct((e_pad, H), f32),
                   jax.ShapeDtypeStruct((e_pad, H), f32)],
    )(ee_spread, g_den, dst2d)


def _tc_softmax_div_spread(ee_spread, g_den, src2d, bm=1024):
    e_pad = ee_spread.shape[0]
    grid = e_pad // bm

    def body(ee_ref, d_ref, src_ref, a_ref, idx_ref):
        ee = jnp.sum(ee_ref[...].reshape(bm, 16, H), axis=1)
        a = ee / d_ref[...][:, :H]
        src = src_ref[...]
        group = lax.broadcasted_iota(i32, (bm, 128), 1) // 8
        a_ref[...] = jnp.tile(a, (1, 16)) * (group == (src % 16)).astype(f32)
        idx_ref[...] = src // 16

    return pl.pallas_call(
        body, grid=(grid,),
        in_specs=[pl.BlockSpec((bm, 128), lambda i: (i, 0)),
                  pl.BlockSpec((bm, 128), lambda i: (i, 0)),
                  pl.BlockSpec((bm, 1), lambda i: (i, 0))],
        out_specs=[pl.BlockSpec((bm, 128), lambda i: (i, 0)),
                   pl.BlockSpec((bm, 1), lambda i: (i, 0))],
        out_shape=[jax.ShapeDtypeStruct((e_pad, 128), f32),
                   jax.ShapeDtypeStruct((e_pad, 1), i32)],
    )(ee_spread, g_den, src2d)


def _tc_final(bp, h2, hs36, ro_w1t, ro_b1, ro_w2t, ro_b2):
    grid = 20
    bn = NL_PAD // grid
    bh = N_GG // grid

    def body(b0_ref, b1_ref, h2_ref, hs_ref, w1_ref, b1v_ref, w2_ref, b2v_ref,
             out_ref, s1, s2):
        i = pl.program_id(0)

        @pl.when(i == 0)
        def _():
            s1[...] = jnp.zeros((1, 128), f32)
            s2[...] = jnp.zeros((1, 36), f32)

        b = b0_ref[...] + b1_ref[...]
        h2b = h2_ref[...].reshape(bn, H, 128)
        contrib = jnp.sum(b[:, :, None] * h2b, axis=(0, 1))
        s1[...] = s1[...] + contrib.reshape(1, 128)
        s2[...] = s2[...] + jnp.sum(hs_ref[...], axis=0).reshape(1, 36)

        @pl.when(i == grid - 1)
        def _():
            y1 = s1[0, :] * (1.0 / N_LG)
            y2p = s2[0, :] * (1.0 / N_GG)
            y = jnp.concatenate([y1, y2p])
            x = jax.nn.sigmoid(jnp.dot(y, w1_ref[...],
                                       preferred_element_type=f32) + b1v_ref[0, :])
            o = jax.nn.sigmoid(jnp.dot(x, w2_ref[...],
                                       preferred_element_type=f32)[0] + b2v_ref[0, 0])
            out_ref[...] = jnp.full((8, 128), o, f32)

    nl_blocks = NL_PAD // bn
    return pl.pallas_call(
        body, grid=(grid,),
        in_specs=[
            pl.BlockSpec((bn, H), lambda i: (i, 0)),
            pl.BlockSpec((bn, H), lambda i: (i + nl_blocks, 0)),
            pl.BlockSpec((bn, 1024), lambda i: (i, 0)),
            pl.BlockSpec((bh, 36), lambda i: (i, 0)),
            pl.BlockSpec((164, 128), lambda i: (0, 0)),
            pl.BlockSpec((1, 128), lambda i: (0, 0)),
            pl.BlockSpec((128, 1), lambda i: (0, 0)),
            pl.BlockSpec((1, 1), lambda i: (0, 0)),
        ],
        out_specs=pl.BlockSpec((8, 128), lambda i: (0, 0)),
        out_shape=jax.ShapeDtypeStruct((8, 128), f32),
        scratch_shapes=[pltpu.VMEM((1, 128), f32), pltpu.VMEM((1, 36), f32)],
    )(bp, bp, h2, hs36, ro_w1t, ro_b1, ro_w2t, ro_b2)


_PERM0 = np.add.outer(np.arange(H) * 128, np.arange(64)).reshape(-1)
_PERM1 = _PERM0 + 64
_P36_PAD = np.concatenate([np.tile(np.eye(36, dtype=np.float32), (H, 1)),
                           np.zeros((96, 36), np.float32)], axis=0)
_A1_ROWS = np.arange(288)
_A1_COLS = np.repeat(np.arange(H), 36)
_A2_ROWS = np.arange(1024)
_A2_COLS = np.repeat(np.arange(H), 128)


def kernel(gg_nfeat, gg_efeat, gg_edge_index, lg_nfeat, lg_efeat, lg_edge_index,
           l1_Wn, l1_bn, l1_Wni, l1_Wfij, l1_Wnj, l1_attn, l1_bias,
           l2_Wn, l2_bn, l2_Wni, l2_Wfij, l2_Wnj, l2_attn, l2_bias,
           ro_W1, ro_b1, ro_W2, ro_b2):
    src_g = jnp.concatenate([gg_edge_index[0], jnp.zeros(EG_PAD - E_GG, i32)])
    dst_g = jnp.concatenate([gg_edge_index[1], jnp.zeros(EG_PAD - E_GG, i32)])
    src_l = jnp.concatenate([lg_edge_index[0], jnp.zeros(EL_PAD - E_LG, i32)])
    dst_l = jnp.concatenate([lg_edge_index[1], jnp.zeros(EL_PAD - E_LG, i32)])
    gg_nf = jnp.concatenate([gg_nfeat, jnp.zeros((NG_PAD - N_GG, 128), f32)])
    lg_nf = jnp.concatenate([lg_nfeat, jnp.zeros((NL_PAD - N_LG, 128), f32)])
    gg_ef = jnp.concatenate([gg_efeat, jnp.zeros((EG_PAD - E_GG, 16), f32)])
    lg_ef = jnp.concatenate([lg_efeat, jnp.zeros((EL_PAD - E_LG, 128), f32)])

    wn1c0_t = l1_Wn[_PERM0, :].T
    wn1c1_t = l1_Wn[_PERM1, :].T
    bn1c0 = l1_bn[_PERM0].reshape(1, 512)
    bn1c1 = l1_bn[_PERM1].reshape(1, 512)
    a1 = jnp.zeros((384, H), f32).at[_A1_ROWS, _A1_COLS].set(l1_attn.reshape(-1))
    a2 = jnp.zeros((1024, H), f32).at[_A2_ROWS, _A2_COLS].set(l2_attn.reshape(-1))
    zero96 = jnp.zeros((1, 96), f32)
    zero1024 = jnp.zeros((1, 1024), f32)
    wni1_t = jnp.concatenate([l1_Wni.T, jnp.zeros((128, 96), f32)], axis=1)
    wnj1_t = jnp.concatenate([l1_Wnj.T, jnp.zeros((128, 96), f32)], axis=1)
    wfij1_t = jnp.concatenate([l1_Wfij.T, jnp.zeros((16, 96), f32)], axis=1)
    bias1 = jnp.concatenate([l1_bias.reshape(1, 288), zero96], axis=1)

    f_ni1, f_nj1, h1c0, h1c1 = _tc_matmul_multi(
        gg_nf,
        [(wni1_t, bias1),
         (wnj1_t, jnp.zeros((1, 384), f32)),
         (wn1c0_t, bn1c0),
         (wn1c1_t, bn1c1)],
        bm=512)

    dst_g2d = dst_g.reshape(-1, 1)
    g_ni1, g_nj1 = _sc_gather2(f_ni1, f_nj1, src_g, dst_g, 64)
    ee1s, idx16_g, hs36 = _tc_edge_gg(
        g_ni1, g_nj1, gg_ef, dst_g2d, wfij1_t, a1, jnp.asarray(_P36_PAD))

    n16g = NG_PAD // 16
    den1p = _sc_scatter_add(ee1s, idx16_g.reshape(-1), n16g, 128)
    den1_8 = jnp.concatenate([den1p[:n16g].reshape(NG_PAD, H),
                              den1p[n16g:].reshape(NG_PAD, H)], axis=0)
    den1_pad = _tc_combine_pad(den1_8, NG_PAD)
    g_den1 = _sc_gather(den1_pad, dst_g, 128)
    afl1, afr1 = _tc_softmax_div(ee1s, g_den1, dst_g2d)

    nhg = NG_PAD // 2
    y2c0p = _sc_wsum(h1c0, afl1.reshape(-1), afr1.reshape(-1),
                     src_g, dst_g, nhg, 32)
    y2c1p = _sc_wsum(h1c1, afl1.reshape(-1), afr1.reshape(-1),
                     src_g, dst_g, nhg, 32)
    y2c0 = jnp.concatenate([y2c0p[:nhg].reshape(NG_PAD, 64),
                            y2c0p[nhg:].reshape(NG_PAD, 64)], axis=0)
    y2c1 = jnp.concatenate([y2c1p[:nhg].reshape(NG_PAD, 64),
                            y2c1p[nhg:].reshape(NG_PAD, 64)], axis=0)

    f_ni2, f_nj2, h2 = _tc_matmul_multi(
        lg_nf,
        [(l2_Wni.T, l2_bias.reshape(1, 1024)),
         (l2_Wnj.T, zero1024),
         (l2_Wn.T, l2_bn.reshape(1, 1024))],
        bm=512)

    dst_l2d = dst_l.reshape(-1, 1)
    src_l2d = src_l.reshape(-1, 1)
    g_ni2 = _sc_gather(f_ni2, src_l, 32)
    g_nj2 = _sc_gather(f_nj2, dst_l, 32)
    ee2s, idx16_l = _tc_edge_lg(lg_ef, y2c0, y2c1, g_ni2, g_nj2, dst_l2d,
                                l2_Wfij.T, a2)

    n16l = NL_PAD // 16
    den2p = _sc_scatter_add(ee2s, idx16_l.reshape(-1), n16l, 128)
    den2_8 = jnp.concatenate([den2p[:n16l].reshape(NL_PAD, H),
                              den2p[n16l:].reshape(NL_PAD, H)], axis=0)
    den2_pad = _tc_combine_pad(den2_8, NL_PAD)
    g_den2 = _sc_gather(den2_pad, dst_l, 128)
    a2s, idx16_s = _tc_softmax_div_spread(ee2s, g_den2, src_l2d)

    bp128 = _sc_scatter_add(a2s, idx16_s.reshape(-1), n16l, 128)
    bp = jnp.concatenate([bp128[:n16l].reshape(NL_PAD, H),
                          bp128[n16l:].reshape(NL_PAD, H)], axis=0)

    out_pad = _tc_final(bp, h2, hs36, ro_W1.T, ro_b1.reshape(1, 128),
                        ro_W2.T, ro_b2.reshape(1, 1))
    return out_pad[0, 0:1]
